# baseline (speedup 1.0000x reference)
import jax
import jax.numpy as jnp
from jax.experimental import pallas as pl

B, N, F, H, C = 128, 512, 256, 256, 10


BB = 4


def _gcn_body(x_ref, adj_ref, w1_ref, b1_ref, w2_ref, b2_ref, out_ref):
    for i in range(BB):
        a = adj_ref[i].astype(jnp.bfloat16)
        h = jnp.dot(x_ref[i].astype(jnp.bfloat16),
                    w1_ref[...].astype(jnp.bfloat16),
                    preferred_element_type=jnp.float32)
        h = h + b1_ref[...]
        h = jnp.dot(a, h.astype(jnp.bfloat16), preferred_element_type=jnp.float32)
        h = jnp.maximum(h, 0.0)
        h = jnp.dot(h.astype(jnp.bfloat16), w2_ref[...].astype(jnp.bfloat16),
                    preferred_element_type=jnp.float32)
        h = h + b2_ref[...]
        out_ref[i] = jnp.dot(a, h.astype(jnp.bfloat16),
                             preferred_element_type=jnp.float32)


def kernel(x, adj, W1, b1, W2, b2):
    b1r = b1.reshape(1, H)
    b2r = b2.reshape(1, C)
    out = pl.pallas_call(
        _gcn_body,
        grid=(B // BB,),
        in_specs=[
            pl.BlockSpec((BB, N, F), lambda b: (b, 0, 0)),
            pl.BlockSpec((BB, N, N), lambda b: (b, 0, 0)),
            pl.BlockSpec((F, H), lambda b: (0, 0)),
            pl.BlockSpec((1, H), lambda b: (0, 0)),
            pl.BlockSpec((H, C), lambda b: (0, 0)),
            pl.BlockSpec((1, C), lambda b: (0, 0)),
        ],
        out_specs=pl.BlockSpec((BB, N, C), lambda b: (b, 0, 0)),
        out_shape=jax.ShapeDtypeStruct((B, N, C), jnp.float32),
    )(x, adj, W1, b1r, W2, b2r)
    return out[None]


# speedup vs baseline: 1.1062x; 1.1062x over previous
import jax
import jax.numpy as jnp
from jax.experimental import pallas as pl

B, N, F, H, C = 128, 512, 256, 256, 10


BB = 8


def _gcn_body(x_ref, adj_ref, w1_ref, b1_ref, w2_ref, b2_ref, out_ref):
    for i in range(BB):
        a = adj_ref[i].astype(jnp.bfloat16)
        h = jnp.dot(x_ref[i].astype(jnp.bfloat16),
                    w1_ref[...].astype(jnp.bfloat16),
                    preferred_element_type=jnp.float32)
        h = h + b1_ref[...]
        h = jnp.dot(a, h.astype(jnp.bfloat16), preferred_element_type=jnp.float32)
        h = jnp.maximum(h, 0.0)
        h = jnp.dot(h.astype(jnp.bfloat16), w2_ref[...].astype(jnp.bfloat16),
                    preferred_element_type=jnp.float32)
        h = h + b2_ref[...]
        out_ref[i] = jnp.dot(a, h.astype(jnp.bfloat16),
                             preferred_element_type=jnp.float32)


def kernel(x, adj, W1, b1, W2, b2):
    b1r = b1.reshape(1, H)
    b2r = b2.reshape(1, C)
    out = pl.pallas_call(
        _gcn_body,
        grid=(B // BB,),
        in_specs=[
            pl.BlockSpec((BB, N, F), lambda b: (b, 0, 0)),
            pl.BlockSpec((BB, N, N), lambda b: (b, 0, 0)),
            pl.BlockSpec((F, H), lambda b: (0, 0)),
            pl.BlockSpec((1, H), lambda b: (0, 0)),
            pl.BlockSpec((H, C), lambda b: (0, 0)),
            pl.BlockSpec((1, C), lambda b: (0, 0)),
        ],
        out_specs=pl.BlockSpec((BB, N, C), lambda b: (b, 0, 0)),
        out_shape=jax.ShapeDtypeStruct((B, N, C), jnp.float32),
    )(x, adj, W1, b1r, W2, b2r)
    return out[None]


# speedup vs baseline: 1.1459x; 1.0359x over previous
import jax
import jax.numpy as jnp
from jax.experimental import pallas as pl
from jax.experimental.pallas import tpu as pltpu

B, N, F, H, C = 128, 512, 256, 256, 10


BB = 16


def _gcn_body(x_ref, adj_ref, w1_ref, b1_ref, w2_ref, b2_ref, out_ref):
    for i in range(BB):
        a = adj_ref[i].astype(jnp.bfloat16)
        h = jnp.dot(x_ref[i].astype(jnp.bfloat16),
                    w1_ref[...].astype(jnp.bfloat16),
                    preferred_element_type=jnp.float32)
        h = h + b1_ref[...]
        h = jnp.dot(a, h.astype(jnp.bfloat16), preferred_element_type=jnp.float32)
        h = jnp.maximum(h, 0.0)
        h = jnp.dot(h.astype(jnp.bfloat16), w2_ref[...].astype(jnp.bfloat16),
                    preferred_element_type=jnp.float32)
        h = h + b2_ref[...]
        out_ref[i] = jnp.dot(a, h.astype(jnp.bfloat16),
                             preferred_element_type=jnp.float32)


def kernel(x, adj, W1, b1, W2, b2):
    b1r = b1.reshape(1, H)
    b2r = b2.reshape(1, C)
    out = pl.pallas_call(
        _gcn_body,
        grid=(B // BB,),
        in_specs=[
            pl.BlockSpec((BB, N, F), lambda b: (b, 0, 0)),
            pl.BlockSpec((BB, N, N), lambda b: (b, 0, 0)),
            pl.BlockSpec((F, H), lambda b: (0, 0)),
            pl.BlockSpec((1, H), lambda b: (0, 0)),
            pl.BlockSpec((H, C), lambda b: (0, 0)),
            pl.BlockSpec((1, C), lambda b: (0, 0)),
        ],
        out_specs=pl.BlockSpec((BB, N, C), lambda b: (b, 0, 0)),
        out_shape=jax.ShapeDtypeStruct((B, N, C), jnp.float32),
        compiler_params=pltpu.CompilerParams(dimension_semantics=("parallel",)),
    )(x, adj, W1, b1r, W2, b2r)
    return out[None]
